# baseline (speedup 1.0000x reference)
import dataclasses

import jax
import jax.numpy as jnp
import numpy as np
from jax import lax
from jax.experimental import pallas as pl
from jax.experimental.pallas import tpu as pltpu
from jax.experimental.pallas import tpu_sc as plsc

EMBED = 256
HEADS = 8
LEVELS = 4
POINTS = 4
DH = EMBED // HEADS
SHAPES_ = [(64, 64), (32, 32), (16, 16), (8, 8)]
NV_ = sum(h * w for h, w in SHAPES_)
BS_ = 2
NQ_ = NV_

QB = 680

_lane = np.arange(128)
_lvl = (_lane % 16) // 4
_head = _lane // 16
_Wl = np.array([w for (_h, w) in SHAPES_], np.int32)[_lvl]
_Hl = np.array([h for (h, _w) in SHAPES_], np.int32)[_lvl]
_base = np.array([0] + list(np.cumsum([h * w for h, w in SHAPES_])[:-1]),
                 np.int64)[_lvl]
_A_np = (_base * 8 + _head).astype(np.int32).reshape(1, 128)
_W8_np = (_Wl * 8).astype(np.int32).reshape(1, 128)
_Wm1_np = (_Wl - 1).astype(np.int32).reshape(1, 128)
_Hm1_np = (_Hl - 1).astype(np.int32).reshape(1, 128)

_PxW_np = np.zeros((8, 128), np.float32)
_PyH_np = np.zeros((8, 128), np.float32)
for _l in range(4):
    _PxW_np[2 * _l, _lvl == _l] = float([w for (_h, w) in SHAPES_][_l])
    _PyH_np[2 * _l + 1, _lvl == _l] = float([h for (h, _w) in SHAPES_][_l])

_G_np = (( _lane[:, None] // 16) == (_lane[None, :] // 16)).astype(np.float32)

_LC_np = np.concatenate([_A_np, _W8_np, _Wm1_np, _Hm1_np], axis=0)

_PERM = np.concatenate([np.arange(0, 256, 2), np.arange(1, 256, 2)])

_VPERM = np.concatenate([np.arange(0, 256, 2), np.arange(1, 256, 2)])

_OPERM = np.concatenate(
    [np.concatenate([h * 32 + np.arange(0, 32, 2),
                     h * 32 + np.arange(1, 32, 2)]) for h in range(8)])


def _prep_kernel(q_ref, v_ref, r8_ref, woff_ref, boff_ref, wattn_ref,
                 battn_ref, wval_ref, bval_ref, pxw_ref, pyh_ref, g_ref,
                 lc_ref, ve_ref, vo_ref, idx_ref, w_ref):
    b = pl.program_id(0)
    q = q_ref[0]
    off = jnp.dot(q, woff_ref[...], preferred_element_type=jnp.float32,
                 precision=lax.Precision.HIGHEST)
    off = off + boff_ref[...]
    offx = off[:, :128]
    offy = off[:, 128:]
    al = jnp.dot(q, wattn_ref[...], preferred_element_type=jnp.float32,
                 precision=lax.Precision.HIGHEST)
    al = al + battn_ref[...]
    e = jnp.exp(al)
    aw = e / jnp.dot(e, g_ref[...], preferred_element_type=jnp.float32,
                 precision=lax.Precision.HIGHEST)
    r8 = r8_ref[0]
    xb = jnp.dot(r8, pxw_ref[...], preferred_element_type=jnp.float32,
                 precision=lax.Precision.HIGHEST)
    yb = jnp.dot(r8, pyh_ref[...], preferred_element_type=jnp.float32,
                 precision=lax.Precision.HIGHEST)
    x = xb + offx - 0.5
    y = yb + offy - 0.5
    x0f = jnp.floor(x)
    y0f = jnp.floor(y)
    fx = x - x0f
    fy = y - y0f
    ix0 = x0f.astype(jnp.int32)
    iy0 = y0f.astype(jnp.int32)

    A = lc_ref[0:1, :]
    W8 = lc_ref[1:2, :]
    Wm1 = lc_ref[2:3, :]
    Hm1 = lc_ref[3:4, :]
    bNV8 = b * (NV_ * 8)

    idx_cs = []
    w_cs = []
    for c, (dy, dx) in enumerate([(0, 0), (0, 1), (1, 0), (1, 1)]):
        ix = ix0 + dx
        iy = iy0 + dy
        vx = ((ix >= 0) & (ix <= Wm1)).astype(jnp.float32)
        vy = ((iy >= 0) & (iy <= Hm1)).astype(jnp.float32)
        wxy = (fx if dx else 1.0 - fx) * (fy if dy else 1.0 - fy)
        wc = aw * wxy * vx * vy
        ixc = jnp.clip(ix, 0, Wm1)
        iyc = jnp.clip(iy, 0, Hm1)
        idx_cs.append(bNV8 + A + iyc * W8 + ixc * 8)
        w_cs.append(wc)
    idx_ref[...] = jnp.stack(idx_cs, axis=1).reshape(QB * 4, 128)
    w_ref[...] = jnp.stack(w_cs, axis=1).reshape(QB * 4, 128)
    vp = (jnp.dot(v_ref[0], wval_ref[...],
                  preferred_element_type=jnp.float32,
                  precision=lax.Precision.HIGHEST) + bval_ref[...])
    ve_ref[0] = vp[:, :128]
    vo_ref[0] = vp[:, 128:]


def _prep(query, value, ref8, woff_p, boff_p, wattn, battn, wval, bval):
    grid = (BS_, NQ_ // QB)
    full = lambda s: pl.BlockSpec(s, lambda b, qb: (0,) * len(s))
    return pl.pallas_call(
        _prep_kernel,
        grid=grid,
        in_specs=[
            pl.BlockSpec((1, QB, 256), lambda b, qb: (b, qb, 0)),
            pl.BlockSpec((1, QB, 256), lambda b, qb: (b, qb, 0)),
            pl.BlockSpec((1, QB, 8), lambda b, qb: (b, qb, 0)),
            full((256, 256)),
            full((1, 256)),
            full((256, 128)),
            full((1, 128)),
            full((256, 256)),
            full((1, 256)),
            full((8, 128)),
            full((8, 128)),
            full((128, 128)),
            full((4, 128)),
        ],
        out_specs=[
            pl.BlockSpec((1, QB, 128), lambda b, qb: (b, qb, 0)),
            pl.BlockSpec((1, QB, 128), lambda b, qb: (b, qb, 0)),
            pl.BlockSpec((QB * 4, 128), lambda b, qb: (b * 8 + qb, 0)),
            pl.BlockSpec((QB * 4, 128), lambda b, qb: (b * 8 + qb, 0)),
        ],
        out_shape=[
            jax.ShapeDtypeStruct((BS_, NQ_, 128), jnp.float32),
            jax.ShapeDtypeStruct((BS_, NQ_, 128), jnp.float32),
            jax.ShapeDtypeStruct((BS_ * NQ_ * 4, 128), jnp.int32),
            jax.ShapeDtypeStruct((BS_ * NQ_ * 4, 128), jnp.float32),
        ],
    )(query, value, ref8, woff_p, boff_p, wattn, battn, wval, bval,
      jnp.asarray(_PxW_np), jnp.asarray(_PyH_np), jnp.asarray(_G_np),
      jnp.asarray(_LC_np))


def _post_kernel(lo_ref, hi_ref, wout_ref, bout_ref, o_ref):
    x = jnp.concatenate([lo_ref[0], hi_ref[0]], axis=-1)
    o_ref[0] = (jnp.dot(x, wout_ref[...],
                        preferred_element_type=jnp.float32,
                 precision=lax.Precision.HIGHEST) + bout_ref[...])


def _post(lo, hi, wout, bout):
    grid = (BS_, NQ_ // QB)
    return pl.pallas_call(
        _post_kernel,
        grid=grid,
        in_specs=[
            pl.BlockSpec((1, QB, 128), lambda b, qb: (b, qb, 0)),
            pl.BlockSpec((1, QB, 128), lambda b, qb: (b, qb, 0)),
            pl.BlockSpec((256, 256), lambda b, qb: (0, 0)),
            pl.BlockSpec((1, 256), lambda b, qb: (0, 0)),
        ],
        out_specs=pl.BlockSpec((1, QB, 256), lambda b, qb: (b, qb, 0)),
        out_shape=jax.ShapeDtypeStruct((BS_, NQ_, 256), jnp.float32),
    )(lo, hi, wout, bout.reshape(1, 256))


NWORK = 32
QTOT = BS_ * NQ_
QPW = QTOT // NWORK
CQ = 2
NSTEP = QPW // CQ
RPQ = 512
NG = CQ * 4


def _sc_body(te_hbm, to_hbm, idx_hbm, w_hbm, lo_hbm, hi_hbm, idx_v0, idx_v1,
             w_v0, w_v1, re_v0, re_v1, ro_v0, ro_v1, lo_v0, lo_v1, hi_v0,
             hi_v1, sem0, sem1, semo0, semo1):
    cid = lax.axis_index("c")
    sid = lax.axis_index("s")
    wid = sid * 2 + cid

    def issue(s, idx_v, w_v, re_v, ro_v, sem):
        q0 = wid * QPW + s * CQ
        pltpu.sync_copy(idx_hbm.at[pl.ds(q0 * 4, NG)], idx_v)
        pltpu.async_copy(w_hbm.at[pl.ds(q0 * 4, NG)], w_v, sem)
        for j in range(NG):
            pltpu.async_copy(te_hbm.at[idx_v.at[j]],
                             re_v.at[pl.ds(j * 128, 128)], sem)
            pltpu.async_copy(to_hbm.at[idx_v.at[j]],
                             ro_v.at[pl.ds(j * 128, 128)], sem)

    def drain(idx_v, w_v, re_v, ro_v, sem):
        q0 = wid * QPW
        pltpu.make_async_copy(w_hbm.at[pl.ds(q0 * 4, NG)], w_v, sem).wait()
        for j in range(NG):
            pltpu.make_async_copy(te_hbm.at[idx_v.at[j]],
                                  re_v.at[pl.ds(j * 128, 128)], sem).wait()
            pltpu.make_async_copy(to_hbm.at[idx_v.at[j]],
                                  ro_v.at[pl.ds(j * 128, 128)], sem).wait()

    def compute(s, w_v, re_v, ro_v, lo_v, hi_v, semo):
        q0 = wid * QPW + s * CQ

        @pl.when(s >= 2)
        def _():
            q0p = q0 - 2 * CQ
            pltpu.make_async_copy(lo_v, lo_hbm.at[pl.ds(q0p, CQ)],
                                  semo).wait()
            pltpu.make_async_copy(hi_v, hi_hbm.at[pl.ds(q0p, CQ)],
                                  semo).wait()

        for h in range(HEADS):
            o_v = lo_v if h < 4 else hi_v
            ob = (h % 4) * 32

            @plsc.parallel_loop(0, CQ, unroll=2)
            def _q(qq, h=h, o_v=o_v, ob=ob):
                pa0 = []
                pa1 = []
                for c in range(4):
                    acc0 = jnp.zeros((16,), jnp.float32)
                    acc1 = jnp.zeros((16,), jnp.float32)
                    w16 = w_v[qq * 4 + c, pl.ds(h * 16, 16)]
                    for k in range(16):
                        p = (qq * 4 + c) * 128 + h * 16 + k
                        wk = lax.gather(
                            w16, jnp.full((16, 1), k, jnp.int32),
                            lax.GatherDimensionNumbers(
                                offset_dims=(), collapsed_slice_dims=(0,),
                                start_index_map=(0,)),
                            (1,),
                            mode=lax.GatherScatterMode.PROMISE_IN_BOUNDS)
                        acc0 = acc0 + wk * re_v[p]
                        acc1 = acc1 + wk * ro_v[p]
                    pa0.append(acc0)
                    pa1.append(acc1)
                o_v[qq, pl.ds(ob, 16)] = (pa0[0] + pa0[1]) + (pa0[2] + pa0[3])
                o_v[qq, pl.ds(ob + 16, 16)] = (
                    (pa1[0] + pa1[1]) + (pa1[2] + pa1[3]))

        pltpu.async_copy(lo_v, lo_hbm.at[pl.ds(q0, CQ)], semo)
        pltpu.async_copy(hi_v, hi_hbm.at[pl.ds(q0, CQ)], semo)

    issue(0, idx_v0, w_v0, re_v0, ro_v0, sem0)

    @pl.loop(0, NSTEP // 2)
    def _g(g):
        s0 = g * 2
        issue(s0 + 1, idx_v1, w_v1, re_v1, ro_v1, sem1)
        drain(idx_v0, w_v0, re_v0, ro_v0, sem0)
        compute(s0, w_v0, re_v0, ro_v0, lo_v0, hi_v0, semo0)

        @pl.when(s0 + 2 < NSTEP)
        def _():
            issue(s0 + 2, idx_v0, w_v0, re_v0, ro_v0, sem0)

        drain(idx_v1, w_v1, re_v1, ro_v1, sem1)
        compute(s0 + 1, w_v1, re_v1, ro_v1, lo_v1, hi_v1, semo1)

    qL0 = (wid * QPW + (NSTEP - 2) * CQ)
    qL1 = (wid * QPW + (NSTEP - 1) * CQ)
    pltpu.make_async_copy(lo_v0, lo_hbm.at[pl.ds(qL0, CQ)], semo0).wait()
    pltpu.make_async_copy(hi_v0, hi_hbm.at[pl.ds(qL0, CQ)], semo0).wait()
    pltpu.make_async_copy(lo_v1, lo_hbm.at[pl.ds(qL1, CQ)], semo1).wait()
    pltpu.make_async_copy(hi_v1, hi_hbm.at[pl.ds(qL1, CQ)], semo1).wait()


def _sc_sample(table_e, table_o, idx2d, w2d):
    mesh = plsc.VectorSubcoreMesh(core_axis_name="c", subcore_axis_name="s")
    cp = pltpu.CompilerParams()
    if "needs_layout_passes" in pltpu.CompilerParams.__dataclass_fields__:
        cp = dataclasses.replace(cp, needs_layout_passes=False)
    if "use_tc_tiling_on_sc" in pltpu.CompilerParams.__dataclass_fields__:
        cp = dataclasses.replace(cp, use_tc_tiling_on_sc=False)
    k = pl.kernel(
        _sc_body,
        mesh=mesh,
        compiler_params=cp,
        out_type=[
            jax.ShapeDtypeStruct((QTOT, 128), jnp.float32),
            jax.ShapeDtypeStruct((QTOT, 128), jnp.float32),
        ],
        scratch_types=[
            pltpu.VMEM((NG, 128), jnp.int32),
            pltpu.VMEM((NG, 128), jnp.int32),
            pltpu.VMEM((NG, 128), jnp.float32),
            pltpu.VMEM((NG, 128), jnp.float32),
            pltpu.VMEM((CQ * RPQ, 16), jnp.float32),
            pltpu.VMEM((CQ * RPQ, 16), jnp.float32),
            pltpu.VMEM((CQ * RPQ, 16), jnp.float32),
            pltpu.VMEM((CQ * RPQ, 16), jnp.float32),
            pltpu.VMEM((CQ, 128), jnp.float32),
            pltpu.VMEM((CQ, 128), jnp.float32),
            pltpu.VMEM((CQ, 128), jnp.float32),
            pltpu.VMEM((CQ, 128), jnp.float32),
            pltpu.SemaphoreType.DMA,
            pltpu.SemaphoreType.DMA,
            pltpu.SemaphoreType.DMA,
            pltpu.SemaphoreType.DMA,
        ],
    )
    return k(table_e, table_o, idx2d, w2d)


def kernel(query, value, reference_points, spatial_shapes, W_off, b_off,
           W_attn, b_attn, W_val, b_val, W_out, b_out):
    del spatial_shapes
    woff_p = W_off[:, jnp.asarray(_PERM)]
    boff_p = b_off[jnp.asarray(_PERM)].reshape(1, 256)
    wval_p = W_val[:, jnp.asarray(_VPERM)]
    bval_p = b_val[jnp.asarray(_VPERM)].reshape(1, 256)
    wout_p = W_out[jnp.asarray(_OPERM), :]
    ref8 = reference_points.reshape(BS_, NQ_, 8)

    ve, vo, idx2d, w2d = _prep(query, value, ref8, woff_p, boff_p, W_attn,
                               b_attn.reshape(1, 128), wval_p, bval_p)
    lo, hi = _sc_sample(ve.reshape(BS_ * NV_ * 8, 16),
                        vo.reshape(BS_ * NV_ * 8, 16), idx2d, w2d)
    return _post(lo.reshape(BS_, NQ_, 128), hi.reshape(BS_, NQ_, 128),
                 wout_p, b_out)


# speedup vs baseline: 2.4969x; 2.4969x over previous
import dataclasses

import jax
import jax.numpy as jnp
import numpy as np
from jax import lax
from jax.experimental import pallas as pl
from jax.experimental.pallas import tpu as pltpu
from jax.experimental.pallas import tpu_sc as plsc

EMBED = 256
HEADS = 8
LEVELS = 4
POINTS = 4
DH = EMBED // HEADS
SHAPES_ = [(64, 64), (32, 32), (16, 16), (8, 8)]
NV_ = sum(h * w for h, w in SHAPES_)
BS_ = 2
NQ_ = NV_

QB = 680

_lane = np.arange(128)
_lvl = (_lane % 16) // 4
_head = _lane // 16
_Wl = np.array([w for (_h, w) in SHAPES_], np.int32)[_lvl]
_Hl = np.array([h for (h, _w) in SHAPES_], np.int32)[_lvl]
_base = np.array([0] + list(np.cumsum([h * w for h, w in SHAPES_])[:-1]),
                 np.int64)[_lvl]
_A_np = (_base * 8 + _head).astype(np.int32).reshape(1, 128)
_W8_np = (_Wl * 8).astype(np.int32).reshape(1, 128)
_Wm1_np = (_Wl - 1).astype(np.int32).reshape(1, 128)
_Hm1_np = (_Hl - 1).astype(np.int32).reshape(1, 128)

_PxW_np = np.zeros((8, 128), np.float32)
_PyH_np = np.zeros((8, 128), np.float32)
for _l in range(4):
    _PxW_np[2 * _l, _lvl == _l] = float([w for (_h, w) in SHAPES_][_l])
    _PyH_np[2 * _l + 1, _lvl == _l] = float([h for (h, _w) in SHAPES_][_l])

_G_np = ((_lane[:, None] // 16) == (_lane[None, :] // 16)).astype(np.float32)

_LC_np = np.concatenate([_A_np, _W8_np, _Wm1_np, _Hm1_np], axis=0)

_PERM = np.concatenate([np.arange(0, 256, 2), np.arange(1, 256, 2)])
_VPERM = np.concatenate([np.arange(0, 256, 2), np.arange(1, 256, 2)])
_OPERM = np.concatenate(
    [np.concatenate([h * 32 + np.arange(0, 32, 2),
                     h * 32 + np.arange(1, 32, 2)]) for h in range(8)])


def _dot3(a, b):
    ah = a.astype(jnp.bfloat16).astype(jnp.float32)
    al = a - ah
    bh = b.astype(jnp.bfloat16).astype(jnp.float32)
    bl = b - bh
    d = lambda x, y: jnp.dot(x, y, preferred_element_type=jnp.float32)
    return d(ah, bh) + (d(ah, bl) + d(al, bh))


def _prep_kernel(q_ref, v_ref, r8_ref, woff_ref, boff_ref, wattn_ref,
                 battn_ref, wval_ref, bval_ref, pxw_ref, pyh_ref, g_ref,
                 lc_ref, vproj_ref, idx_ref, w_ref):
    b = pl.program_id(0)
    q = q_ref[0]
    off = _dot3(q, woff_ref[...])
    off = off + boff_ref[...]
    offx = off[:, :128]
    offy = off[:, 128:]
    al = _dot3(q, wattn_ref[...])
    al = al + battn_ref[...]
    e = jnp.exp(al)
    aw = e / _dot3(e, g_ref[...])
    r8 = r8_ref[0]
    xb = jnp.dot(r8, pxw_ref[...], preferred_element_type=jnp.float32,
                 precision=lax.Precision.HIGHEST)
    yb = jnp.dot(r8, pyh_ref[...], preferred_element_type=jnp.float32,
                 precision=lax.Precision.HIGHEST)
    x = xb + offx - 0.5
    y = yb + offy - 0.5
    x0f = jnp.floor(x)
    y0f = jnp.floor(y)
    fx = x - x0f
    fy = y - y0f
    ix0 = x0f.astype(jnp.int32)
    iy0 = y0f.astype(jnp.int32)

    A = lc_ref[0:1, :]
    W8 = lc_ref[1:2, :]
    Wm1 = lc_ref[2:3, :]
    Hm1 = lc_ref[3:4, :]
    bNV8 = b * (NV_ * 8)

    for c, (dy, dx) in enumerate([(0, 0), (0, 1), (1, 0), (1, 1)]):
        ix = ix0 + dx
        iy = iy0 + dy
        vx = ((ix >= 0) & (ix <= Wm1)).astype(jnp.float32)
        vy = ((iy >= 0) & (iy <= Hm1)).astype(jnp.float32)
        wxy = (fx if dx else 1.0 - fx) * (fy if dy else 1.0 - fy)
        wc = aw * wxy * vx * vy
        ixc = jnp.clip(ix, 0, Wm1)
        iyc = jnp.clip(iy, 0, Hm1)
        idx_c = bNV8 + A + iyc * W8 + ixc * 8
        idx_ref[0, :, c * 128:(c + 1) * 128] = idx_c
        w_ref[0, :, c * 128:(c + 1) * 128] = wc
    vp = _dot3(v_ref[0], wval_ref[...]) + bval_ref[...]
    be = lax.bitcast_convert_type(vp[:, :128], jnp.int32)
    bo = lax.bitcast_convert_type(vp[:, 128:], jnp.int32)

    def _rnd(bits):
        return bits + 0x7FFF + (lax.shift_right_logical(bits, 16) & 1)

    ue = lax.shift_right_logical(_rnd(be), 16)
    uo = _rnd(bo) & jnp.int32(-65536)
    vproj_ref[0] = ue | uo


def _prep(query, value, ref8, woff_p, boff_p, wattn, battn, wval, bval):
    grid = (BS_, NQ_ // QB)
    full = lambda s: pl.BlockSpec(s, lambda b, qb: (0,) * len(s))
    return pl.pallas_call(
        _prep_kernel,
        grid=grid,
        in_specs=[
            pl.BlockSpec((1, QB, 256), lambda b, qb: (b, qb, 0)),
            pl.BlockSpec((1, QB, 256), lambda b, qb: (b, qb, 0)),
            pl.BlockSpec((1, QB, 8), lambda b, qb: (b, qb, 0)),
            full((256, 256)),
            full((1, 256)),
            full((256, 128)),
            full((1, 128)),
            full((256, 256)),
            full((1, 256)),
            full((8, 128)),
            full((8, 128)),
            full((128, 128)),
            full((4, 128)),
        ],
        out_specs=[
            pl.BlockSpec((1, QB, 128), lambda b, qb: (b, qb, 0)),
            pl.BlockSpec((1, QB, 512), lambda b, qb: (b, qb, 0)),
            pl.BlockSpec((1, QB, 512), lambda b, qb: (b, qb, 0)),
        ],
        out_shape=[
            jax.ShapeDtypeStruct((BS_, NQ_, 128), jnp.int32),
            jax.ShapeDtypeStruct((BS_, NQ_, 512), jnp.int32),
            jax.ShapeDtypeStruct((BS_, NQ_, 512), jnp.float32),
        ],
    )(query, value, ref8, woff_p, boff_p, wattn, battn, wval, bval,
      jnp.asarray(_PxW_np), jnp.asarray(_PyH_np), jnp.asarray(_G_np),
      jnp.asarray(_LC_np))


def _post_kernel(s_ref, wout_ref, bout_ref, o_ref):
    o_ref[0] = _dot3(s_ref[0], wout_ref[...]) + bout_ref[...]


def _post(sampled, wout, bout):
    grid = (BS_, NQ_ // QB)
    return pl.pallas_call(
        _post_kernel,
        grid=grid,
        in_specs=[
            pl.BlockSpec((1, QB, 256), lambda b, qb: (b, qb, 0)),
            pl.BlockSpec((256, 256), lambda b, qb: (0, 0)),
            pl.BlockSpec((1, 256), lambda b, qb: (0, 0)),
        ],
        out_specs=pl.BlockSpec((1, QB, 256), lambda b, qb: (b, qb, 0)),
        out_shape=jax.ShapeDtypeStruct((BS_, NQ_, 256), jnp.float32),
    )(sampled, wout, bout.reshape(1, 256))


NWORK = 32
QTOT = BS_ * NQ_
QPW = QTOT // NWORK
CQ = 5
NSTEP = QPW // CQ
RPQ = 512
NG = CQ * 4


def _sc_body(table_hbm, idx_hbm, w_hbm, out_hbm, idx_v0, idx_v1, w_v0, w_v1,
             rows_v0, rows_v1, out_v, sem0, sem1):
    cid = lax.axis_index("c")
    sid = lax.axis_index("s")
    wid = sid * 2 + cid

    def issue(s, idx_v, w_v, rows_v, sem):
        q0 = wid * QPW + s * CQ
        pltpu.sync_copy(idx_hbm.at[pl.ds(q0 * 4, NG)], idx_v)
        pltpu.sync_copy(w_hbm.at[pl.ds(q0 * RPQ, CQ * RPQ)], w_v)
        for j in range(NG):
            pltpu.async_copy(table_hbm.at[idx_v.at[j]],
                             rows_v.at[pl.ds(j * 128, 128)], sem)

    def drain(idx_v, rows_v, sem):
        for j in range(NG):
            pltpu.make_async_copy(table_hbm.at[idx_v.at[j]],
                                  rows_v.at[pl.ds(j * 128, 128)], sem).wait()

    def compute(s, w_v, rows_v):
        q0 = wid * QPW + s * CQ

        @plsc.parallel_loop(0, CQ * HEADS, unroll=2)
        def _qh(t):
            qq = t // HEADS
            h = t % HEADS
            base = qq * RPQ + h * 16
            pa0 = []
            pa1 = []
            for c in range(4):
                acc0 = jnp.zeros((16,), jnp.float32)
                acc1 = jnp.zeros((16,), jnp.float32)
                w16 = w_v[pl.ds(base + c * 128, 16)]
                for k in range(16):
                    p = base + c * 128 + k
                    wk = lax.gather(
                        w16, jnp.full((16, 1), k, jnp.int32),
                        lax.GatherDimensionNumbers(
                            offset_dims=(), collapsed_slice_dims=(0,),
                            start_index_map=(0,)),
                        (1,),
                        mode=lax.GatherScatterMode.PROMISE_IN_BOUNDS)
                    r = rows_v[p]
                    bf = plsc.bitcast(r, jnp.bfloat16)
                    re, ro = plsc.unpack(bf, format=plsc.PackFormat.INTERLEAVED)
                    acc0 = acc0 + wk * re
                    acc1 = acc1 + wk * ro
                pa0.append(acc0)
                pa1.append(acc1)
            o = qq * 256 + h * 32
            out_v[pl.ds(o, 16)] = (pa0[0] + pa0[1]) + (pa0[2] + pa0[3])
            out_v[pl.ds(o + 16, 16)] = (pa1[0] + pa1[1]) + (pa1[2] + pa1[3])

        pltpu.sync_copy(out_v, out_hbm.at[pl.ds(q0 * 256, CQ * 256)])

    issue(0, idx_v0, w_v0, rows_v0, sem0)

    @pl.loop(0, NSTEP // 2)
    def _g(g):
        s0 = g * 2
        issue(s0 + 1, idx_v1, w_v1, rows_v1, sem1)
        drain(idx_v0, rows_v0, sem0)
        compute(s0, w_v0, rows_v0)

        @pl.when(s0 + 2 < NSTEP)
        def _():
            issue(s0 + 2, idx_v0, w_v0, rows_v0, sem0)

        drain(idx_v1, rows_v1, sem1)
        compute(s0 + 1, w_v1, rows_v1)


def _sc_sample(table, idx2d, w1d):
    mesh = plsc.VectorSubcoreMesh(core_axis_name="c", subcore_axis_name="s")
    cp = pltpu.CompilerParams()
    if "needs_layout_passes" in pltpu.CompilerParams.__dataclass_fields__:
        cp = dataclasses.replace(cp, needs_layout_passes=False)
    if "use_tc_tiling_on_sc" in pltpu.CompilerParams.__dataclass_fields__:
        cp = dataclasses.replace(cp, use_tc_tiling_on_sc=False)
    k = pl.kernel(
        _sc_body,
        mesh=mesh,
        compiler_params=cp,
        out_type=jax.ShapeDtypeStruct((QTOT * 256,), jnp.float32),
        scratch_types=[
            pltpu.VMEM((NG, 128), jnp.int32),
            pltpu.VMEM((NG, 128), jnp.int32),
            pltpu.VMEM((CQ * RPQ,), jnp.float32),
            pltpu.VMEM((CQ * RPQ,), jnp.float32),
            pltpu.VMEM((CQ * RPQ, 16), jnp.int32),
            pltpu.VMEM((CQ * RPQ, 16), jnp.int32),
            pltpu.VMEM((CQ * 256,), jnp.float32),
            pltpu.SemaphoreType.DMA,
            pltpu.SemaphoreType.DMA,
        ],
    )
    return k(table, idx2d, w1d)


def kernel(query, value, reference_points, spatial_shapes, W_off, b_off,
           W_attn, b_attn, W_val, b_val, W_out, b_out):
    del spatial_shapes
    woff_p = W_off[:, jnp.asarray(_PERM)]
    boff_p = b_off[jnp.asarray(_PERM)].reshape(1, 256)
    wval_p = W_val[:, jnp.asarray(_VPERM)]
    bval_p = b_val[jnp.asarray(_VPERM)].reshape(1, 256)
    wout_p = W_out[jnp.asarray(_OPERM), :]
    ref8 = reference_points.reshape(BS_, NQ_, 8)

    vpack, idx, wts = _prep(query, value, ref8, woff_p, boff_p, W_attn,
                            b_attn.reshape(1, 128), wval_p, bval_p)
    table = vpack.reshape(BS_ * NV_ * 8, 16)
    idx2d = idx.reshape(QTOT * 4, 128)
    w1d = wts.reshape(QTOT * RPQ)
    sampled = _sc_sample(table, idx2d, w1d)
    return _post(sampled.reshape(BS_, NQ_, 256), wout_p, b_out)


# speedup vs baseline: 2.5004x; 1.0014x over previous
import dataclasses

import jax
import jax.numpy as jnp
import numpy as np
from jax import lax
from jax.experimental import pallas as pl
from jax.experimental.pallas import tpu as pltpu
from jax.experimental.pallas import tpu_sc as plsc

EMBED = 256
HEADS = 8
LEVELS = 4
POINTS = 4
DH = EMBED // HEADS
SHAPES_ = [(64, 64), (32, 32), (16, 16), (8, 8)]
NV_ = sum(h * w for h, w in SHAPES_)
BS_ = 2
NQ_ = NV_

QB = 680

_lane = np.arange(128)
_lvl = (_lane % 16) // 4
_head = _lane // 16
_Wl = np.array([w for (_h, w) in SHAPES_], np.int32)[_lvl]
_Hl = np.array([h for (h, _w) in SHAPES_], np.int32)[_lvl]
_base = np.array([0] + list(np.cumsum([h * w for h, w in SHAPES_])[:-1]),
                 np.int64)[_lvl]
_A_np = (_base * 8 + _head).astype(np.int32).reshape(1, 128)
_W8_np = (_Wl * 8).astype(np.int32).reshape(1, 128)
_Wm1_np = (_Wl - 1).astype(np.int32).reshape(1, 128)
_Hm1_np = (_Hl - 1).astype(np.int32).reshape(1, 128)

_PxW_np = np.zeros((8, 128), np.float32)
_PyH_np = np.zeros((8, 128), np.float32)
for _l in range(4):
    _PxW_np[2 * _l, _lvl == _l] = float([w for (_h, w) in SHAPES_][_l])
    _PyH_np[2 * _l + 1, _lvl == _l] = float([h for (h, _w) in SHAPES_][_l])

_G_np = ((_lane[:, None] // 16) == (_lane[None, :] // 16)).astype(np.float32)

_LC_np = np.concatenate([_A_np, _W8_np, _Wm1_np, _Hm1_np], axis=0)

_PERM = np.concatenate([np.arange(0, 256, 2), np.arange(1, 256, 2)])
_VPERM = np.concatenate([np.arange(0, 256, 2), np.arange(1, 256, 2)])
_OPERM = np.concatenate(
    [np.concatenate([h * 32 + np.arange(0, 32, 2),
                     h * 32 + np.arange(1, 32, 2)]) for h in range(8)])


def _dot3(a, b):
    ah = a.astype(jnp.bfloat16).astype(jnp.float32)
    al = a - ah
    bh = b.astype(jnp.bfloat16).astype(jnp.float32)
    bl = b - bh
    d = lambda x, y: jnp.dot(x, y, preferred_element_type=jnp.float32)
    return d(ah, bh) + (d(ah, bl) + d(al, bh))


def _prep_kernel(q_ref, v_ref, r8_ref, woff_ref, boff_ref, wattn_ref,
                 battn_ref, wval_ref, bval_ref, pxw_ref, pyh_ref, g_ref,
                 lc_ref, vproj_ref, idx_ref, w_ref):
    b = pl.program_id(0)
    q = q_ref[0]
    off = _dot3(q, woff_ref[...])
    off = off + boff_ref[...]
    offx = off[:, :128]
    offy = off[:, 128:]
    al = _dot3(q, wattn_ref[...])
    al = al + battn_ref[...]
    e = jnp.exp(al)
    aw = e / _dot3(e, g_ref[...])
    r8 = r8_ref[0]
    xb = jnp.dot(r8, pxw_ref[...], preferred_element_type=jnp.float32,
                 precision=lax.Precision.HIGHEST)
    yb = jnp.dot(r8, pyh_ref[...], preferred_element_type=jnp.float32,
                 precision=lax.Precision.HIGHEST)
    x = xb + offx - 0.5
    y = yb + offy - 0.5
    x0f = jnp.floor(x)
    y0f = jnp.floor(y)
    fx = x - x0f
    fy = y - y0f
    ix0 = x0f.astype(jnp.int32)
    iy0 = y0f.astype(jnp.int32)

    A = lc_ref[0:1, :]
    W8 = lc_ref[1:2, :]
    Wm1 = lc_ref[2:3, :]
    Hm1 = lc_ref[3:4, :]
    bNV8 = b * (NV_ * 8)

    for c, (dy, dx) in enumerate([(0, 0), (0, 1), (1, 0), (1, 1)]):
        ix = ix0 + dx
        iy = iy0 + dy
        vx = ((ix >= 0) & (ix <= Wm1)).astype(jnp.float32)
        vy = ((iy >= 0) & (iy <= Hm1)).astype(jnp.float32)
        wxy = (fx if dx else 1.0 - fx) * (fy if dy else 1.0 - fy)
        wc = aw * wxy * vx * vy
        ixc = jnp.clip(ix, 0, Wm1)
        iyc = jnp.clip(iy, 0, Hm1)
        idx_c = bNV8 + A + iyc * W8 + ixc * 8
        idx_ref[0, :, c * 128:(c + 1) * 128] = idx_c
        w_ref[0, :, c * 128:(c + 1) * 128] = wc
    vp = _dot3(v_ref[0], wval_ref[...]) + bval_ref[...]
    be = lax.bitcast_convert_type(vp[:, :128], jnp.int32)
    bo = lax.bitcast_convert_type(vp[:, 128:], jnp.int32)

    def _rnd(bits):
        return bits + 0x7FFF + (lax.shift_right_logical(bits, 16) & 1)

    ue = lax.shift_right_logical(_rnd(be), 16)
    uo = _rnd(bo) & jnp.int32(-65536)
    vproj_ref[0] = ue | uo


def _prep(query, value, ref8, woff_p, boff_p, wattn, battn, wval, bval):
    grid = (BS_, NQ_ // QB)
    full = lambda s: pl.BlockSpec(s, lambda b, qb: (0,) * len(s))
    return pl.pallas_call(
        _prep_kernel,
        grid=grid,
        in_specs=[
            pl.BlockSpec((1, QB, 256), lambda b, qb: (b, qb, 0)),
            pl.BlockSpec((1, QB, 256), lambda b, qb: (b, qb, 0)),
            pl.BlockSpec((1, QB, 8), lambda b, qb: (b, qb, 0)),
            full((256, 256)),
            full((1, 256)),
            full((256, 128)),
            full((1, 128)),
            full((256, 256)),
            full((1, 256)),
            full((8, 128)),
            full((8, 128)),
            full((128, 128)),
            full((4, 128)),
        ],
        out_specs=[
            pl.BlockSpec((1, QB, 128), lambda b, qb: (b, qb, 0)),
            pl.BlockSpec((1, QB, 512), lambda b, qb: (b, qb, 0)),
            pl.BlockSpec((1, QB, 512), lambda b, qb: (b, qb, 0)),
        ],
        out_shape=[
            jax.ShapeDtypeStruct((BS_, NQ_, 128), jnp.int32),
            jax.ShapeDtypeStruct((BS_, NQ_, 512), jnp.int32),
            jax.ShapeDtypeStruct((BS_, NQ_, 512), jnp.float32),
        ],
    )(query, value, ref8, woff_p, boff_p, wattn, battn, wval, bval,
      jnp.asarray(_PxW_np), jnp.asarray(_PyH_np), jnp.asarray(_G_np),
      jnp.asarray(_LC_np))


def _post_kernel(s_ref, wout_ref, bout_ref, o_ref):
    o_ref[0] = _dot3(s_ref[0], wout_ref[...]) + bout_ref[...]


def _post(sampled, wout, bout):
    grid = (BS_, NQ_ // QB)
    return pl.pallas_call(
        _post_kernel,
        grid=grid,
        in_specs=[
            pl.BlockSpec((1, QB, 256), lambda b, qb: (b, qb, 0)),
            pl.BlockSpec((256, 256), lambda b, qb: (0, 0)),
            pl.BlockSpec((1, 256), lambda b, qb: (0, 0)),
        ],
        out_specs=pl.BlockSpec((1, QB, 256), lambda b, qb: (b, qb, 0)),
        out_shape=jax.ShapeDtypeStruct((BS_, NQ_, 256), jnp.float32),
    )(sampled, wout, bout.reshape(1, 256))


NWORK = 32
QTOT = BS_ * NQ_
QPW = QTOT // NWORK
CQ = 5
NSTEP = QPW // CQ
RPQ = 512
NG = CQ * 4


def _sc_body(table_hbm, idx_hbm, w_hbm, out_hbm, idx_v0, idx_v1, w_v0, w_v1,
             rows_v0, rows_v1, out_v0, out_v1, sem0, sem1, semo0, semo1):
    cid = lax.axis_index("c")
    sid = lax.axis_index("s")
    wid = sid * 2 + cid

    def issue(s, idx_v, w_v, rows_v, sem):
        q0 = wid * QPW + s * CQ
        pltpu.sync_copy(idx_hbm.at[pl.ds(q0 * 4, NG)], idx_v)
        pltpu.async_copy(w_hbm.at[pl.ds(q0 * RPQ, CQ * RPQ)], w_v, sem)
        for j in range(NG):
            pltpu.async_copy(table_hbm.at[idx_v.at[j]],
                             rows_v.at[pl.ds(j * 128, 128)], sem)

    def drain(idx_v, w_v, rows_v, sem):
        pltpu.make_async_copy(w_hbm.at[pl.ds(0, CQ * RPQ)], w_v, sem).wait()
        for j in range(NG):
            pltpu.make_async_copy(table_hbm.at[idx_v.at[j]],
                                  rows_v.at[pl.ds(j * 128, 128)], sem).wait()

    def compute(s, w_v, rows_v, out_v, semo):
        q0 = wid * QPW + s * CQ

        @pl.when(s >= 2)
        def _():
            pltpu.make_async_copy(
                out_v, out_hbm.at[pl.ds((q0 - 2 * CQ) * 256, CQ * 256)],
                semo).wait()

        @plsc.parallel_loop(0, CQ * HEADS, unroll=2)
        def _qh(t):
            qq = t // HEADS
            h = t % HEADS
            base = qq * RPQ + h * 16
            pa0 = []
            pa1 = []
            for c in range(4):
                acc0 = jnp.zeros((16,), jnp.float32)
                acc1 = jnp.zeros((16,), jnp.float32)
                w16 = w_v[pl.ds(base + c * 128, 16)]
                for k in range(16):
                    p = base + c * 128 + k
                    wk = lax.gather(
                        w16, jnp.full((16, 1), k, jnp.int32),
                        lax.GatherDimensionNumbers(
                            offset_dims=(), collapsed_slice_dims=(0,),
                            start_index_map=(0,)),
                        (1,),
                        mode=lax.GatherScatterMode.PROMISE_IN_BOUNDS)
                    r = rows_v[p]
                    bf = plsc.bitcast(r, jnp.bfloat16)
                    re, ro = plsc.unpack(bf, format=plsc.PackFormat.INTERLEAVED)
                    acc0 = acc0 + wk * re
                    acc1 = acc1 + wk * ro
                pa0.append(acc0)
                pa1.append(acc1)
            o = qq * 256 + h * 32
            out_v[pl.ds(o, 16)] = (pa0[0] + pa0[1]) + (pa0[2] + pa0[3])
            out_v[pl.ds(o + 16, 16)] = (pa1[0] + pa1[1]) + (pa1[2] + pa1[3])

        pltpu.async_copy(out_v, out_hbm.at[pl.ds(q0 * 256, CQ * 256)], semo)

    issue(0, idx_v0, w_v0, rows_v0, sem0)

    @pl.loop(0, NSTEP // 2)
    def _g(g):
        s0 = g * 2
        issue(s0 + 1, idx_v1, w_v1, rows_v1, sem1)
        drain(idx_v0, w_v0, rows_v0, sem0)
        compute(s0, w_v0, rows_v0, out_v0, semo0)

        @pl.when(s0 + 2 < NSTEP)
        def _():
            issue(s0 + 2, idx_v0, w_v0, rows_v0, sem0)

        drain(idx_v1, w_v1, rows_v1, sem1)
        compute(s0 + 1, w_v1, rows_v1, out_v1, semo1)

    qL0 = (wid * QPW + (NSTEP - 2) * CQ) * 256
    qL1 = (wid * QPW + (NSTEP - 1) * CQ) * 256
    pltpu.make_async_copy(out_v0, out_hbm.at[pl.ds(qL0, CQ * 256)],
                          semo0).wait()
    pltpu.make_async_copy(out_v1, out_hbm.at[pl.ds(qL1, CQ * 256)],
                          semo1).wait()


def _sc_sample(table, idx2d, w1d):
    mesh = plsc.VectorSubcoreMesh(core_axis_name="c", subcore_axis_name="s")
    cp = pltpu.CompilerParams()
    if "needs_layout_passes" in pltpu.CompilerParams.__dataclass_fields__:
        cp = dataclasses.replace(cp, needs_layout_passes=False)
    if "use_tc_tiling_on_sc" in pltpu.CompilerParams.__dataclass_fields__:
        cp = dataclasses.replace(cp, use_tc_tiling_on_sc=False)
    k = pl.kernel(
        _sc_body,
        mesh=mesh,
        compiler_params=cp,
        out_type=jax.ShapeDtypeStruct((QTOT * 256,), jnp.float32),
        scratch_types=[
            pltpu.VMEM((NG, 128), jnp.int32),
            pltpu.VMEM((NG, 128), jnp.int32),
            pltpu.VMEM((CQ * RPQ,), jnp.float32),
            pltpu.VMEM((CQ * RPQ,), jnp.float32),
            pltpu.VMEM((CQ * RPQ, 16), jnp.int32),
            pltpu.VMEM((CQ * RPQ, 16), jnp.int32),
            pltpu.VMEM((CQ * 256,), jnp.float32),
            pltpu.VMEM((CQ * 256,), jnp.float32),
            pltpu.SemaphoreType.DMA,
            pltpu.SemaphoreType.DMA,
            pltpu.SemaphoreType.DMA,
            pltpu.SemaphoreType.DMA,
        ],
    )
    return k(table, idx2d, w1d)


def kernel(query, value, reference_points, spatial_shapes, W_off, b_off,
           W_attn, b_attn, W_val, b_val, W_out, b_out):
    del spatial_shapes
    woff_p = W_off[:, jnp.asarray(_PERM)]
    boff_p = b_off[jnp.asarray(_PERM)].reshape(1, 256)
    wval_p = W_val[:, jnp.asarray(_VPERM)]
    bval_p = b_val[jnp.asarray(_VPERM)].reshape(1, 256)
    wout_p = W_out[jnp.asarray(_OPERM), :]
    ref8 = reference_points.reshape(BS_, NQ_, 8)

    vpack, idx, wts = _prep(query, value, ref8, woff_p, boff_p, W_attn,
                            b_attn.reshape(1, 128), wval_p, bval_p)
    table = vpack.reshape(BS_ * NV_ * 8, 16)
    idx2d = idx.reshape(QTOT * 4, 128)
    w1d = wts.reshape(QTOT * RPQ)
    sampled = _sc_sample(table, idx2d, w1d)
    return _post(sampled.reshape(BS_, NQ_, 256), wout_p, b_out)
